# baseline (speedup 1.0000x reference)
import functools

import jax
import jax.numpy as jnp
from jax import lax
from jax.experimental import pallas as pl
from jax.experimental.pallas import tpu as pltpu
from jax.experimental.pallas import tpu_sc as plsc

B, Q, S, Cq, Cs = 8, 4096, 1024, 128, 256
D0, D1, D2 = Cq + Cs, 256, 128
QT = 256
NQT = Q // QT
N_BN = B * Q

NW = 32
QPW = (B * Q) // NW
SUBQ = 64
NCH = QPW // SUBQ


def _kernel_a1(qpt_ref, sp_ref, di_ref, dw_ref):
    b = pl.program_id(0)

    qpt = qpt_ref[0]
    sp = sp_ref[0]
    qq = jnp.sum(qpt * qpt, axis=1, keepdims=True)
    ss = jnp.sum(sp * sp, axis=0, keepdims=True)
    qs = lax.dot_general(qpt, sp, (((1,), (0,)), ((), ())),
                         preferred_element_type=jnp.float32)
    d2 = qq + ss - 2.0 * qs

    inf = jnp.float32(jnp.inf)
    m1 = jnp.min(d2, axis=1, keepdims=True)
    m2 = jnp.min(jnp.where(d2 > m1, d2, inf), axis=1, keepdims=True)
    m3 = jnp.min(jnp.where(d2 > m2, d2, inf), axis=1, keepdims=True)

    iota_col = lax.broadcasted_iota(jnp.int32, (S, 1), 0).astype(jnp.float32)
    gbase = (b * S).astype(jnp.float32)
    iks = [lax.dot_general(jnp.where(d2 == m, 1.0, 0.0), iota_col,
                           (((1,), (0,)), ((), ())),
                           precision=lax.Precision.HIGHEST,
                           preferred_element_type=jnp.float32)
           + (gbase + 0.5)
           for m in (m1, m2, m3)]

    ws = [1.0 / (jnp.maximum(m, 0.0) + 1e-5) for m in (m1, m2, m3)]
    wsum = ws[0] + ws[1] + ws[2]
    ws = [w / wsum for w in ws]

    di_ref[0] = jnp.concatenate(iks, axis=1).astype(jnp.int32)
    dw_ref[...] = jnp.concatenate(
        [jnp.broadcast_to(w, (QT, 16)) for w in ws], axis=1)


def _kernel_a0(sf_ref, w1_ref, pt_ref):
    pt_ref[...] = lax.dot_general(
        sf_ref[0], w1_ref[:, Cq:], (((0,), (1,)), ((), ())),
        preferred_element_type=jnp.float32)


def _sc_body(pt_hbm, di_hbm, dw_hbm, z_hbm,
             idx0, idx1, w0_, r0, r1, o0,
             si0, si1, sw0, sg0, sg1, so0):
    c = lax.axis_index("c")
    s = lax.axis_index("s")
    wid = s * 2 + c
    n0 = wid * QPW
    idxb, rb = [idx0, idx1], [r0, r1]
    sib, sgb = [si0, si1], [sg0, sg1]

    def start_idx(ch, sl):
        row0 = n0 + ch * SUBQ
        return pltpu.async_copy(di_hbm.at[pl.ds(3 * row0, 3 * SUBQ)],
                                idxb[sl], sib[sl])

    def start_w(ch):
        row0 = n0 + ch * SUBQ
        return pltpu.async_copy(dw_hbm.at[pl.ds(row0, SUBQ)], w0_, sw0)

    def compute(sl):
        rows_v = rb[sl]

        def body(q, carry):
            base = 3 * q
            wv0 = w0_[q, pl.ds(0, 16)]
            wv1 = w0_[q, pl.ds(16, 16)]
            wv2 = w0_[q, pl.ds(32, 16)]
            for j in range(D1 // 16):
                slc = pl.ds(16 * j, 16)
                o0[q, slc] = (
                    wv0 * rows_v[base, slc] + wv1 * rows_v[base + 1, slc]
                    + wv2 * rows_v[base + 2, slc])
            return carry

        lax.fori_loop(0, SUBQ, body, 0)

    hidx = [None, None]
    hg = [None, None]
    hout = None
    hidx[0] = start_idx(0, 0)
    hidx[1] = start_idx(1, 1)
    hw = start_w(0)
    hidx[0].wait()
    hg[0] = pltpu.async_copy(pt_hbm.at[idxb[0]], rb[0], sgb[0])
    for ch in range(NCH):
        sl = ch % 2
        if ch + 1 < NCH:
            hidx[1 - sl].wait()
            hg[1 - sl] = pltpu.async_copy(pt_hbm.at[idxb[1 - sl]],
                                          rb[1 - sl], sgb[1 - sl])
        hg[sl].wait()
        hw.wait()
        if hout is not None:
            hout.wait()
        compute(sl)
        hout = pltpu.async_copy(o0, z_hbm.at[pl.ds(n0 + ch * SUBQ, SUBQ)],
                                so0)
        if ch + 2 < NCH:
            hidx[sl] = start_idx(ch + 2, sl)
        if ch + 1 < NCH:
            hw = start_w(ch + 1)
    hout.wait()


def _interp_gather(pt, di_flat, dw_splat):
    mesh = plsc.VectorSubcoreMesh(core_axis_name="c", subcore_axis_name="s")
    return pl.kernel(
        _sc_body,
        mesh=mesh,
        out_type=jax.ShapeDtypeStruct((B * Q, D1), jnp.float32),
        scratch_types=(
            [pltpu.VMEM((3 * SUBQ,), jnp.int32) for _ in range(2)]
            + [pltpu.VMEM((SUBQ, 48), jnp.float32)]
            + [pltpu.VMEM((3 * SUBQ, D1), jnp.float32) for _ in range(2)]
            + [pltpu.VMEM((SUBQ, D1), jnp.float32)]
            + [pltpu.SemaphoreType.DMA for _ in range(6)]
        ),
    )(pt, di_flat, dw_splat)


def _kernel_c1(qf_ref, w1_ref, b1_ref, z_ref, y1_ref, s1_ref, ss1_ref):
    b = pl.program_id(0)
    qt = pl.program_id(1)
    yq = lax.dot_general(w1_ref[:, :Cq], qf_ref[0], (((1,), (0,)), ((), ())),
                         preferred_element_type=jnp.float32)
    y1 = yq + b1_ref[...] + jnp.transpose(z_ref[...])
    y1_ref[0] = y1

    @pl.when(jnp.logical_and(b == 0, qt == 0))
    def _():
        s1_ref[...] = jnp.zeros_like(s1_ref)
        ss1_ref[...] = jnp.zeros_like(ss1_ref)

    s1_ref[...] += jnp.sum(y1, axis=1, keepdims=True)
    ss1_ref[...] += jnp.sum(y1 * y1, axis=1, keepdims=True)


def _kernel_b(y1_ref, s1_ref, ss1_ref, g1_ref, be1_ref, w2_ref, b2_ref,
              y2_ref, s2_ref, ss2_ref):
    b = pl.program_id(0)
    qt = pl.program_id(1)

    mean = s1_ref[...] * (1.0 / N_BN)
    var = ss1_ref[...] * (1.0 / N_BN) - mean * mean
    scale = g1_ref[...] * lax.rsqrt(var + 1e-5)
    shift = be1_ref[...] - mean * scale
    x1 = jnp.maximum(y1_ref[0] * scale + shift, 0.0)
    y2 = lax.dot_general(w2_ref[...], x1, (((1,), (0,)), ((), ())),
                         preferred_element_type=jnp.float32) + b2_ref[...]
    y2_ref[0] = y2

    @pl.when(jnp.logical_and(b == 0, qt == 0))
    def _():
        s2_ref[...] = jnp.zeros_like(s2_ref)
        ss2_ref[...] = jnp.zeros_like(ss2_ref)

    s2_ref[...] += jnp.sum(y2, axis=1, keepdims=True)
    ss2_ref[...] += jnp.sum(y2 * y2, axis=1, keepdims=True)


def _kernel_c(y2_ref, s2_ref, ss2_ref, g2_ref, be2_ref, out_ref):
    mean = s2_ref[...] * (1.0 / N_BN)
    var = ss2_ref[...] * (1.0 / N_BN) - mean * mean
    scale = g2_ref[...] * lax.rsqrt(var + 1e-5)
    shift = be2_ref[...] - mean * scale
    out_ref[0] = jnp.maximum(y2_ref[0] * scale + shift, 0.0)


def kernel(q_points, s_points, q_feats, s_feats, W1, b1, g1, be1,
           W2, b2, g2, be2):
    qpt = q_points.transpose(0, 2, 1)
    b1c = b1.reshape(D1, 1)
    g1c = g1.reshape(D1, 1)
    be1c = be1.reshape(D1, 1)
    b2c = b2.reshape(D2, 1)
    g2c = g2.reshape(D2, 1)
    be2c = be2.reshape(D2, 1)

    col = lambda d: pl.BlockSpec((d, 1), lambda b, q: (0, 0))

    di, dw = pl.pallas_call(
        _kernel_a1,
        grid=(B, NQT),
        in_specs=[
            pl.BlockSpec((1, QT, 3), lambda b, q: (b, q, 0)),
            pl.BlockSpec((1, 3, S), lambda b, q: (b, 0, 0)),
        ],
        out_specs=[
            pl.BlockSpec((1, QT, 3), lambda b, q: (b, q, 0)),
            pl.BlockSpec((QT, 48), lambda b, q: (b * NQT + q, 0)),
        ],
        out_shape=[
            jax.ShapeDtypeStruct((B, Q, 3), jnp.int32),
            jax.ShapeDtypeStruct((B * Q, 48), jnp.float32),
        ],
    )(qpt, s_points)

    pt = pl.pallas_call(
        _kernel_a0,
        grid=(B,),
        in_specs=[
            pl.BlockSpec((1, Cs, S), lambda b: (b, 0, 0)),
            pl.BlockSpec((D1, D0), lambda b: (0, 0)),
        ],
        out_specs=pl.BlockSpec((S, D1), lambda b: (b, 0)),
        out_shape=jax.ShapeDtypeStruct((B * S, D1), jnp.float32),
    )(s_feats, W1)

    z = _interp_gather(pt, di.reshape(-1), dw)

    y1, s1, ss1 = pl.pallas_call(
        _kernel_c1,
        grid=(B, NQT),
        in_specs=[
            pl.BlockSpec((1, Cq, QT), lambda b, q: (b, 0, q)),
            pl.BlockSpec((D1, D0), lambda b, q: (0, 0)),
            col(D1),
            pl.BlockSpec((QT, D1), lambda b, q: (b * NQT + q, 0)),
        ],
        out_specs=[
            pl.BlockSpec((1, D1, QT), lambda b, q: (b, 0, q)),
            col(D1),
            col(D1),
        ],
        out_shape=[
            jax.ShapeDtypeStruct((B, D1, Q), jnp.float32),
            jax.ShapeDtypeStruct((D1, 1), jnp.float32),
            jax.ShapeDtypeStruct((D1, 1), jnp.float32),
        ],
    )(q_feats, W1, b1c, z)

    y2, s2, ss2 = pl.pallas_call(
        _kernel_b,
        grid=(B, NQT),
        in_specs=[
            pl.BlockSpec((1, D1, QT), lambda b, q: (b, 0, q)),
            col(D1), col(D1), col(D1), col(D1),
            pl.BlockSpec((D2, D1), lambda b, q: (0, 0)),
            col(D2),
        ],
        out_specs=[
            pl.BlockSpec((1, D2, QT), lambda b, q: (b, 0, q)),
            col(D2),
            col(D2),
        ],
        out_shape=[
            jax.ShapeDtypeStruct((B, D2, Q), jnp.float32),
            jax.ShapeDtypeStruct((D2, 1), jnp.float32),
            jax.ShapeDtypeStruct((D2, 1), jnp.float32),
        ],
    )(y1, s1, ss1, g1c, be1c, W2, b2c)

    out = pl.pallas_call(
        _kernel_c,
        grid=(B, NQT),
        in_specs=[
            pl.BlockSpec((1, D2, QT), lambda b, q: (b, 0, q)),
            col(D2), col(D2), col(D2), col(D2),
        ],
        out_specs=pl.BlockSpec((1, D2, QT), lambda b, q: (b, 0, q)),
        out_shape=jax.ShapeDtypeStruct((B, D2, Q), jnp.float32),
    )(y2, s2, ss2, g2c, be2c)
    return out


# speedup vs baseline: 1.4245x; 1.4245x over previous
import functools

import jax
import jax.numpy as jnp
from jax import lax
from jax.experimental import pallas as pl
from jax.experimental.pallas import tpu as pltpu
from jax.experimental.pallas import tpu_sc as plsc

B, Q, S, Cq, Cs = 8, 4096, 1024, 128, 256
D0, D1, D2 = Cq + Cs, 256, 128
QT = 256
NQT = Q // QT
N_BN = B * Q

NW = 32
QPW = (B * Q) // NW
SUBQ = 64
NCH = QPW // SUBQ


def _kernel_a1(qpt_ref, sp_ref, di_ref, dw_ref):
    b = pl.program_id(0)

    qpt = qpt_ref[0]
    sp = sp_ref[0]
    qq = jnp.sum(qpt * qpt, axis=1, keepdims=True)
    ss = jnp.sum(sp * sp, axis=0, keepdims=True)
    qs = lax.dot_general(qpt, sp, (((1,), (0,)), ((), ())),
                         preferred_element_type=jnp.float32)
    d2 = qq + ss - 2.0 * qs

    inf = jnp.float32(jnp.inf)
    m1 = jnp.min(d2, axis=1, keepdims=True)
    m2 = jnp.min(jnp.where(d2 > m1, d2, inf), axis=1, keepdims=True)
    m3 = jnp.min(jnp.where(d2 > m2, d2, inf), axis=1, keepdims=True)

    iota = lax.broadcasted_iota(jnp.int32, (QT, S), 1)
    gbase = b * S
    iks = [jnp.min(jnp.where(d2 == m, iota, S), axis=1, keepdims=True) + gbase
           for m in (m1, m2, m3)]

    ws = [1.0 / (jnp.maximum(m, 0.0) + 1e-5) for m in (m1, m2, m3)]
    wsum = ws[0] + ws[1] + ws[2]
    ws = [w / wsum for w in ws]

    di_ref[0] = jnp.concatenate(iks, axis=1)
    dw_ref[...] = jnp.concatenate(
        [jnp.broadcast_to(w, (QT, 16)) for w in ws], axis=1)


def _kernel_a0(sf_ref, w1_ref, pt_ref):
    pt_ref[...] = lax.dot_general(
        sf_ref[0], w1_ref[:, Cq:], (((0,), (1,)), ((), ())),
        preferred_element_type=jnp.float32)


def _sc_body(pt_hbm, di_hbm, dw_hbm, z_hbm,
             idx0, idx1, w0_, r0, r1, o0,
             si0, si1, sw0, sg0, sg1, so0):
    c = lax.axis_index("c")
    s = lax.axis_index("s")
    wid = s * 2 + c
    n0 = wid * QPW
    idxb, rb = [idx0, idx1], [r0, r1]
    sib, sgb = [si0, si1], [sg0, sg1]

    def start_idx(ch, sl):
        row0 = n0 + ch * SUBQ
        return pltpu.async_copy(di_hbm.at[pl.ds(3 * row0, 3 * SUBQ)],
                                idxb[sl], sib[sl])

    def start_w(ch):
        row0 = n0 + ch * SUBQ
        return pltpu.async_copy(dw_hbm.at[pl.ds(row0, SUBQ)], w0_, sw0)

    def compute(sl):
        rows_v = rb[sl]

        def body(q, carry):
            base = 3 * q
            wv0 = w0_[q, pl.ds(0, 16)]
            wv1 = w0_[q, pl.ds(16, 16)]
            wv2 = w0_[q, pl.ds(32, 16)]
            for j in range(D1 // 16):
                slc = pl.ds(16 * j, 16)
                o0[q, slc] = (
                    wv0 * rows_v[base, slc] + wv1 * rows_v[base + 1, slc]
                    + wv2 * rows_v[base + 2, slc])
            return carry

        lax.fori_loop(0, SUBQ, body, 0)

    hidx = [None, None]
    hg = [None, None]
    hout = None
    hidx[0] = start_idx(0, 0)
    hidx[1] = start_idx(1, 1)
    hw = start_w(0)
    hidx[0].wait()
    hg[0] = pltpu.async_copy(pt_hbm.at[idxb[0]], rb[0], sgb[0])
    for ch in range(NCH):
        sl = ch % 2
        if ch + 1 < NCH:
            hidx[1 - sl].wait()
            hg[1 - sl] = pltpu.async_copy(pt_hbm.at[idxb[1 - sl]],
                                          rb[1 - sl], sgb[1 - sl])
        hg[sl].wait()
        hw.wait()
        if hout is not None:
            hout.wait()
        compute(sl)
        hout = pltpu.async_copy(o0, z_hbm.at[pl.ds(n0 + ch * SUBQ, SUBQ)],
                                so0)
        if ch + 2 < NCH:
            hidx[sl] = start_idx(ch + 2, sl)
        if ch + 1 < NCH:
            hw = start_w(ch + 1)
    hout.wait()


def _interp_gather(pt, di_flat, dw_splat):
    mesh = plsc.VectorSubcoreMesh(core_axis_name="c", subcore_axis_name="s")
    return pl.kernel(
        _sc_body,
        mesh=mesh,
        out_type=jax.ShapeDtypeStruct((B * Q, D1), jnp.float32),
        scratch_types=(
            [pltpu.VMEM((3 * SUBQ,), jnp.int32) for _ in range(2)]
            + [pltpu.VMEM((SUBQ, 48), jnp.float32)]
            + [pltpu.VMEM((3 * SUBQ, D1), jnp.float32) for _ in range(2)]
            + [pltpu.VMEM((SUBQ, D1), jnp.float32)]
            + [pltpu.SemaphoreType.DMA for _ in range(6)]
        ),
    )(pt, di_flat, dw_splat)


def _kernel_c1(qf_ref, w1_ref, b1_ref, z_ref, y1_ref, s1_ref, ss1_ref):
    b = pl.program_id(0)
    qt = pl.program_id(1)
    yq = lax.dot_general(w1_ref[:, :Cq], qf_ref[0], (((1,), (0,)), ((), ())),
                         preferred_element_type=jnp.float32)
    y1 = yq + b1_ref[...] + jnp.transpose(z_ref[...])
    y1_ref[0] = y1

    @pl.when(jnp.logical_and(b == 0, qt == 0))
    def _():
        s1_ref[...] = jnp.zeros_like(s1_ref)
        ss1_ref[...] = jnp.zeros_like(ss1_ref)

    s1_ref[...] += jnp.sum(y1, axis=1, keepdims=True)
    ss1_ref[...] += jnp.sum(y1 * y1, axis=1, keepdims=True)


def _kernel_b(y1_ref, s1_ref, ss1_ref, g1_ref, be1_ref, w2_ref, b2_ref,
              y2_ref, s2_ref, ss2_ref):
    b = pl.program_id(0)
    qt = pl.program_id(1)

    mean = s1_ref[...] * (1.0 / N_BN)
    var = ss1_ref[...] * (1.0 / N_BN) - mean * mean
    scale = g1_ref[...] * lax.rsqrt(var + 1e-5)
    shift = be1_ref[...] - mean * scale
    x1 = jnp.maximum(y1_ref[0] * scale + shift, 0.0)
    y2 = lax.dot_general(w2_ref[...], x1, (((1,), (0,)), ((), ())),
                         preferred_element_type=jnp.float32) + b2_ref[...]
    y2_ref[0] = y2

    @pl.when(jnp.logical_and(b == 0, qt == 0))
    def _():
        s2_ref[...] = jnp.zeros_like(s2_ref)
        ss2_ref[...] = jnp.zeros_like(ss2_ref)

    s2_ref[...] += jnp.sum(y2, axis=1, keepdims=True)
    ss2_ref[...] += jnp.sum(y2 * y2, axis=1, keepdims=True)


def _kernel_c(y2_ref, s2_ref, ss2_ref, g2_ref, be2_ref, out_ref):
    mean = s2_ref[...] * (1.0 / N_BN)
    var = ss2_ref[...] * (1.0 / N_BN) - mean * mean
    scale = g2_ref[...] * lax.rsqrt(var + 1e-5)
    shift = be2_ref[...] - mean * scale
    out_ref[0] = jnp.maximum(y2_ref[0] * scale + shift, 0.0)


def kernel(q_points, s_points, q_feats, s_feats, W1, b1, g1, be1,
           W2, b2, g2, be2):
    qpt = q_points.transpose(0, 2, 1)
    b1c = b1.reshape(D1, 1)
    g1c = g1.reshape(D1, 1)
    be1c = be1.reshape(D1, 1)
    b2c = b2.reshape(D2, 1)
    g2c = g2.reshape(D2, 1)
    be2c = be2.reshape(D2, 1)

    col = lambda d: pl.BlockSpec((d, 1), lambda b, q: (0, 0))

    di, dw = pl.pallas_call(
        _kernel_a1,
        grid=(B, NQT),
        in_specs=[
            pl.BlockSpec((1, QT, 3), lambda b, q: (b, q, 0)),
            pl.BlockSpec((1, 3, S), lambda b, q: (b, 0, 0)),
        ],
        out_specs=[
            pl.BlockSpec((1, QT, 3), lambda b, q: (b, q, 0)),
            pl.BlockSpec((QT, 48), lambda b, q: (b * NQT + q, 0)),
        ],
        out_shape=[
            jax.ShapeDtypeStruct((B, Q, 3), jnp.int32),
            jax.ShapeDtypeStruct((B * Q, 48), jnp.float32),
        ],
    )(qpt, s_points)

    pt = pl.pallas_call(
        _kernel_a0,
        grid=(B,),
        in_specs=[
            pl.BlockSpec((1, Cs, S), lambda b: (b, 0, 0)),
            pl.BlockSpec((D1, D0), lambda b: (0, 0)),
        ],
        out_specs=pl.BlockSpec((S, D1), lambda b: (b, 0)),
        out_shape=jax.ShapeDtypeStruct((B * S, D1), jnp.float32),
    )(s_feats, W1)

    z = _interp_gather(pt, di.reshape(-1), dw)

    y1, s1, ss1 = pl.pallas_call(
        _kernel_c1,
        grid=(B, NQT),
        in_specs=[
            pl.BlockSpec((1, Cq, QT), lambda b, q: (b, 0, q)),
            pl.BlockSpec((D1, D0), lambda b, q: (0, 0)),
            col(D1),
            pl.BlockSpec((QT, D1), lambda b, q: (b * NQT + q, 0)),
        ],
        out_specs=[
            pl.BlockSpec((1, D1, QT), lambda b, q: (b, 0, q)),
            col(D1),
            col(D1),
        ],
        out_shape=[
            jax.ShapeDtypeStruct((B, D1, Q), jnp.float32),
            jax.ShapeDtypeStruct((D1, 1), jnp.float32),
            jax.ShapeDtypeStruct((D1, 1), jnp.float32),
        ],
    )(q_feats, W1, b1c, z)

    y2, s2, ss2 = pl.pallas_call(
        _kernel_b,
        grid=(B, NQT),
        in_specs=[
            pl.BlockSpec((1, D1, QT), lambda b, q: (b, 0, q)),
            col(D1), col(D1), col(D1), col(D1),
            pl.BlockSpec((D2, D1), lambda b, q: (0, 0)),
            col(D2),
        ],
        out_specs=[
            pl.BlockSpec((1, D2, QT), lambda b, q: (b, 0, q)),
            col(D2),
            col(D2),
        ],
        out_shape=[
            jax.ShapeDtypeStruct((B, D2, Q), jnp.float32),
            jax.ShapeDtypeStruct((D2, 1), jnp.float32),
            jax.ShapeDtypeStruct((D2, 1), jnp.float32),
        ],
    )(y1, s1, ss1, g1c, be1c, W2, b2c)

    out = pl.pallas_call(
        _kernel_c,
        grid=(B, NQT),
        in_specs=[
            pl.BlockSpec((1, D2, QT), lambda b, q: (b, 0, q)),
            col(D2), col(D2), col(D2), col(D2),
        ],
        out_specs=pl.BlockSpec((1, D2, QT), lambda b, q: (b, 0, q)),
        out_shape=jax.ShapeDtypeStruct((B, D2, Q), jnp.float32),
    )(y2, s2, ss2, g2c, be2c)
    return out


# speedup vs baseline: 1.7240x; 1.2102x over previous
import functools

import jax
import jax.numpy as jnp
from jax import lax
from jax.experimental import pallas as pl
from jax.experimental.pallas import tpu as pltpu
from jax.experimental.pallas import tpu_sc as plsc

B, Q, S, Cq, Cs = 8, 4096, 1024, 128, 256
D0, D1, D2 = Cq + Cs, 256, 128
QT = 256
NQT = Q // QT
QT2 = 512
NQT2 = Q // QT2
N_BN = B * Q

NW = 32
QPW = (B * Q) // NW
SUBQ = 64
NCH = QPW // SUBQ


def _kernel_a1(qpt_ref, sp_ref, di_ref, dw_ref):
    b = pl.program_id(0)

    qpt = qpt_ref[0]
    sp = sp_ref[0]
    qq = jnp.sum(qpt * qpt, axis=1, keepdims=True)
    ss = jnp.sum(sp * sp, axis=0, keepdims=True)
    qs = lax.dot_general(qpt, sp, (((1,), (0,)), ((), ())),
                         preferred_element_type=jnp.float32)
    d2 = qq + ss - 2.0 * qs

    inf = jnp.float32(jnp.inf)
    m1 = jnp.min(d2, axis=1, keepdims=True)
    m2 = jnp.min(jnp.where(d2 > m1, d2, inf), axis=1, keepdims=True)
    m3 = jnp.min(jnp.where(d2 > m2, d2, inf), axis=1, keepdims=True)

    iota = lax.broadcasted_iota(jnp.int32, (QT, S), 1)
    gbase = b * S
    iks = [jnp.min(jnp.where(d2 == m, iota, S), axis=1, keepdims=True) + gbase
           for m in (m1, m2, m3)]

    ws = [1.0 / (jnp.maximum(m, 0.0) + 1e-5) for m in (m1, m2, m3)]
    wsum = ws[0] + ws[1] + ws[2]
    ws = [w / wsum for w in ws]

    di_ref[0] = jnp.concatenate(iks, axis=1)
    dw_ref[...] = jnp.concatenate(
        [jnp.broadcast_to(w, (QT, 16)) for w in ws], axis=1)


def _kernel_a0(sf_ref, w1_ref, pt_ref):
    pt_ref[...] = lax.dot_general(
        sf_ref[0], w1_ref[:, Cq:], (((0,), (1,)), ((), ())),
        preferred_element_type=jnp.float32)


def _sc_body(pt_hbm, di_hbm, dw_hbm, z_hbm,
             idx0, idx1, w0_, r0, r1, o0,
             si0, si1, sw0, sg0, sg1, so0):
    c = lax.axis_index("c")
    s = lax.axis_index("s")
    wid = s * 2 + c
    n0 = wid * QPW
    idxb, rb = [idx0, idx1], [r0, r1]
    sib, sgb = [si0, si1], [sg0, sg1]

    def start_idx(ch, sl):
        row0 = n0 + ch * SUBQ
        return pltpu.async_copy(di_hbm.at[pl.ds(3 * row0, 3 * SUBQ)],
                                idxb[sl], sib[sl])

    def start_w(ch):
        row0 = n0 + ch * SUBQ
        return pltpu.async_copy(dw_hbm.at[pl.ds(row0, SUBQ)], w0_, sw0)

    def compute(sl):
        rows_v = rb[sl]

        def body(q, carry):
            base = 3 * q
            wv0 = w0_[q, pl.ds(0, 16)]
            wv1 = w0_[q, pl.ds(16, 16)]
            wv2 = w0_[q, pl.ds(32, 16)]
            for j in range(D1 // 16):
                slc = pl.ds(16 * j, 16)
                o0[q, slc] = (
                    wv0 * rows_v[base, slc] + wv1 * rows_v[base + 1, slc]
                    + wv2 * rows_v[base + 2, slc])
            return carry

        lax.fori_loop(0, SUBQ, body, 0)

    hidx = [None, None]
    hg = [None, None]
    hout = None
    hidx[0] = start_idx(0, 0)
    hidx[1] = start_idx(1, 1)
    hw = start_w(0)
    hidx[0].wait()
    hg[0] = pltpu.async_copy(pt_hbm.at[idxb[0]], rb[0], sgb[0])
    for ch in range(NCH):
        sl = ch % 2
        if ch + 1 < NCH:
            hidx[1 - sl].wait()
            hg[1 - sl] = pltpu.async_copy(pt_hbm.at[idxb[1 - sl]],
                                          rb[1 - sl], sgb[1 - sl])
        hg[sl].wait()
        hw.wait()
        if hout is not None:
            hout.wait()
        compute(sl)
        hout = pltpu.async_copy(o0, z_hbm.at[pl.ds(n0 + ch * SUBQ, SUBQ)],
                                so0)
        if ch + 2 < NCH:
            hidx[sl] = start_idx(ch + 2, sl)
        if ch + 1 < NCH:
            hw = start_w(ch + 1)
    hout.wait()


def _interp_gather(pt, di_flat, dw_splat):
    mesh = plsc.VectorSubcoreMesh(core_axis_name="c", subcore_axis_name="s")
    return pl.kernel(
        _sc_body,
        mesh=mesh,
        out_type=jax.ShapeDtypeStruct((B * Q, D1), jnp.float32),
        scratch_types=(
            [pltpu.VMEM((3 * SUBQ,), jnp.int32) for _ in range(2)]
            + [pltpu.VMEM((SUBQ, 48), jnp.float32)]
            + [pltpu.VMEM((3 * SUBQ, D1), jnp.float32) for _ in range(2)]
            + [pltpu.VMEM((SUBQ, D1), jnp.float32)]
            + [pltpu.SemaphoreType.DMA for _ in range(6)]
        ),
    )(pt, di_flat, dw_splat)


def _kernel_c1(qf_ref, w1_ref, b1_ref, z_ref, y1_ref, s1_ref, ss1_ref):
    b = pl.program_id(0)
    qt = pl.program_id(1)
    yq = lax.dot_general(w1_ref[:, :Cq], qf_ref[0], (((1,), (0,)), ((), ())),
                         preferred_element_type=jnp.float32)
    y1 = yq + b1_ref[...] + jnp.transpose(z_ref[...])
    y1_ref[0] = y1

    @pl.when(jnp.logical_and(b == 0, qt == 0))
    def _():
        s1_ref[...] = jnp.zeros_like(s1_ref)
        ss1_ref[...] = jnp.zeros_like(ss1_ref)

    s1_ref[...] += jnp.sum(y1, axis=1, keepdims=True)
    ss1_ref[...] += jnp.sum(y1 * y1, axis=1, keepdims=True)


def _kernel_b(y1_ref, s1_ref, ss1_ref, g1_ref, be1_ref, w2_ref, b2_ref,
              y2_ref, s2_ref, ss2_ref):
    b = pl.program_id(0)
    qt = pl.program_id(1)

    mean = s1_ref[...] * (1.0 / N_BN)
    var = ss1_ref[...] * (1.0 / N_BN) - mean * mean
    scale = g1_ref[...] * lax.rsqrt(var + 1e-5)
    shift = be1_ref[...] - mean * scale
    x1 = jnp.maximum(y1_ref[0] * scale + shift, 0.0)
    y2 = lax.dot_general(w2_ref[...], x1, (((1,), (0,)), ((), ())),
                         preferred_element_type=jnp.float32) + b2_ref[...]
    y2_ref[0] = y2

    @pl.when(jnp.logical_and(b == 0, qt == 0))
    def _():
        s2_ref[...] = jnp.zeros_like(s2_ref)
        ss2_ref[...] = jnp.zeros_like(ss2_ref)

    s2_ref[...] += jnp.sum(y2, axis=1, keepdims=True)
    ss2_ref[...] += jnp.sum(y2 * y2, axis=1, keepdims=True)


def _kernel_c(y2_ref, s2_ref, ss2_ref, g2_ref, be2_ref, out_ref):
    mean = s2_ref[...] * (1.0 / N_BN)
    var = ss2_ref[...] * (1.0 / N_BN) - mean * mean
    scale = g2_ref[...] * lax.rsqrt(var + 1e-5)
    shift = be2_ref[...] - mean * scale
    out_ref[0] = jnp.maximum(y2_ref[0] * scale + shift, 0.0)


def kernel(q_points, s_points, q_feats, s_feats, W1, b1, g1, be1,
           W2, b2, g2, be2):
    qpt = q_points.transpose(0, 2, 1)
    b1c = b1.reshape(D1, 1)
    g1c = g1.reshape(D1, 1)
    be1c = be1.reshape(D1, 1)
    b2c = b2.reshape(D2, 1)
    g2c = g2.reshape(D2, 1)
    be2c = be2.reshape(D2, 1)

    col = lambda d: pl.BlockSpec((d, 1), lambda b, q: (0, 0))

    di, dw = pl.pallas_call(
        _kernel_a1,
        grid=(B, NQT),
        in_specs=[
            pl.BlockSpec((1, QT, 3), lambda b, q: (b, q, 0)),
            pl.BlockSpec((1, 3, S), lambda b, q: (b, 0, 0)),
        ],
        out_specs=[
            pl.BlockSpec((1, QT, 3), lambda b, q: (b, q, 0)),
            pl.BlockSpec((QT, 48), lambda b, q: (b * NQT + q, 0)),
        ],
        out_shape=[
            jax.ShapeDtypeStruct((B, Q, 3), jnp.int32),
            jax.ShapeDtypeStruct((B * Q, 48), jnp.float32),
        ],
    )(qpt, s_points)

    pt = pl.pallas_call(
        _kernel_a0,
        grid=(B,),
        in_specs=[
            pl.BlockSpec((1, Cs, S), lambda b: (b, 0, 0)),
            pl.BlockSpec((D1, D0), lambda b: (0, 0)),
        ],
        out_specs=pl.BlockSpec((S, D1), lambda b: (b, 0)),
        out_shape=jax.ShapeDtypeStruct((B * S, D1), jnp.float32),
    )(s_feats, W1)

    z = _interp_gather(pt, di.reshape(-1), dw)

    y1, s1, ss1 = pl.pallas_call(
        _kernel_c1,
        grid=(B, NQT2),
        in_specs=[
            pl.BlockSpec((1, Cq, QT2), lambda b, q: (b, 0, q)),
            pl.BlockSpec((D1, D0), lambda b, q: (0, 0)),
            col(D1),
            pl.BlockSpec((QT2, D1), lambda b, q: (b * NQT2 + q, 0)),
        ],
        out_specs=[
            pl.BlockSpec((1, D1, QT2), lambda b, q: (b, 0, q)),
            col(D1),
            col(D1),
        ],
        out_shape=[
            jax.ShapeDtypeStruct((B, D1, Q), jnp.float32),
            jax.ShapeDtypeStruct((D1, 1), jnp.float32),
            jax.ShapeDtypeStruct((D1, 1), jnp.float32),
        ],
    )(q_feats, W1, b1c, z)

    y2, s2, ss2 = pl.pallas_call(
        _kernel_b,
        grid=(B, NQT2),
        in_specs=[
            pl.BlockSpec((1, D1, QT2), lambda b, q: (b, 0, q)),
            col(D1), col(D1), col(D1), col(D1),
            pl.BlockSpec((D2, D1), lambda b, q: (0, 0)),
            col(D2),
        ],
        out_specs=[
            pl.BlockSpec((1, D2, QT2), lambda b, q: (b, 0, q)),
            col(D2),
            col(D2),
        ],
        out_shape=[
            jax.ShapeDtypeStruct((B, D2, Q), jnp.float32),
            jax.ShapeDtypeStruct((D2, 1), jnp.float32),
            jax.ShapeDtypeStruct((D2, 1), jnp.float32),
        ],
    )(y1, s1, ss1, g1c, be1c, W2, b2c)

    out = pl.pallas_call(
        _kernel_c,
        grid=(B, NQT2),
        in_specs=[
            pl.BlockSpec((1, D2, QT2), lambda b, q: (b, 0, q)),
            col(D2), col(D2), col(D2), col(D2),
        ],
        out_specs=pl.BlockSpec((1, D2, QT2), lambda b, q: (b, 0, q)),
        out_shape=jax.ShapeDtypeStruct((B, D2, Q), jnp.float32),
    )(y2, s2, ss2, g2c, be2c)
    return out


# speedup vs baseline: 1.9385x; 1.1244x over previous
import functools

import jax
import jax.numpy as jnp
from jax import lax
from jax.experimental import pallas as pl
from jax.experimental.pallas import tpu as pltpu
from jax.experimental.pallas import tpu_sc as plsc

B, Q, S, Cq, Cs = 8, 4096, 1024, 128, 256
D0, D1, D2 = Cq + Cs, 256, 128
QT = 256
NQT = Q // QT
QT2 = 1024
NQT2 = Q // QT2
N_BN = B * Q

NW = 32
QPW = (B * Q) // NW
SUBQ = 64
NCH = QPW // SUBQ


def _kernel_a1(qpt_ref, sp_ref, di_ref, dw_ref):
    b = pl.program_id(0)

    qpt = qpt_ref[0]
    sp = sp_ref[0]
    qq = jnp.sum(qpt * qpt, axis=1, keepdims=True)
    ss = jnp.sum(sp * sp, axis=0, keepdims=True)
    qs = lax.dot_general(qpt, sp, (((1,), (0,)), ((), ())),
                         preferred_element_type=jnp.float32)
    d2 = qq + ss - 2.0 * qs

    inf = jnp.float32(jnp.inf)
    m1 = jnp.min(d2, axis=1, keepdims=True)
    m2 = jnp.min(jnp.where(d2 > m1, d2, inf), axis=1, keepdims=True)
    m3 = jnp.min(jnp.where(d2 > m2, d2, inf), axis=1, keepdims=True)

    iota = lax.broadcasted_iota(jnp.int32, (QT, S), 1)
    gbase = b * S
    iks = [jnp.min(jnp.where(d2 == m, iota, S), axis=1, keepdims=True) + gbase
           for m in (m1, m2, m3)]

    ws = [1.0 / (jnp.maximum(m, 0.0) + 1e-5) for m in (m1, m2, m3)]
    wsum = ws[0] + ws[1] + ws[2]
    ws = [w / wsum for w in ws]

    di_ref[0] = jnp.concatenate(iks, axis=1)
    dw_ref[...] = jnp.concatenate(
        [jnp.broadcast_to(w, (QT, 16)) for w in ws], axis=1)


def _kernel_a0(sf_ref, w1_ref, pt_ref):
    pt_ref[...] = lax.dot_general(
        sf_ref[0], w1_ref[:, Cq:], (((0,), (1,)), ((), ())),
        preferred_element_type=jnp.float32)


def _sc_body(pt_hbm, di_hbm, dw_hbm, z_hbm,
             idx0, idx1, w0_, r0, r1, o0,
             si0, si1, sw0, sg0, sg1, so0):
    c = lax.axis_index("c")
    s = lax.axis_index("s")
    wid = s * 2 + c
    n0 = wid * QPW
    idxb, rb = [idx0, idx1], [r0, r1]
    sib, sgb = [si0, si1], [sg0, sg1]

    def start_idx(ch, sl):
        row0 = n0 + ch * SUBQ
        return pltpu.async_copy(di_hbm.at[pl.ds(3 * row0, 3 * SUBQ)],
                                idxb[sl], sib[sl])

    def start_w(ch):
        row0 = n0 + ch * SUBQ
        return pltpu.async_copy(dw_hbm.at[pl.ds(row0, SUBQ)], w0_, sw0)

    def compute(sl):
        rows_v = rb[sl]

        def body(q, carry):
            base = 3 * q
            wv0 = w0_[q, pl.ds(0, 16)]
            wv1 = w0_[q, pl.ds(16, 16)]
            wv2 = w0_[q, pl.ds(32, 16)]
            for j in range(D1 // 16):
                slc = pl.ds(16 * j, 16)
                o0[q, slc] = (
                    wv0 * rows_v[base, slc] + wv1 * rows_v[base + 1, slc]
                    + wv2 * rows_v[base + 2, slc])
            return carry

        lax.fori_loop(0, SUBQ, body, 0)

    hidx = [None, None]
    hg = [None, None]
    hout = None
    hidx[0] = start_idx(0, 0)
    hidx[1] = start_idx(1, 1)
    hw = start_w(0)
    hidx[0].wait()
    hg[0] = pltpu.async_copy(pt_hbm.at[idxb[0]], rb[0], sgb[0])
    for ch in range(NCH):
        sl = ch % 2
        if ch + 1 < NCH:
            hidx[1 - sl].wait()
            hg[1 - sl] = pltpu.async_copy(pt_hbm.at[idxb[1 - sl]],
                                          rb[1 - sl], sgb[1 - sl])
        hg[sl].wait()
        hw.wait()
        if hout is not None:
            hout.wait()
        compute(sl)
        hout = pltpu.async_copy(o0, z_hbm.at[pl.ds(n0 + ch * SUBQ, SUBQ)],
                                so0)
        if ch + 2 < NCH:
            hidx[sl] = start_idx(ch + 2, sl)
        if ch + 1 < NCH:
            hw = start_w(ch + 1)
    hout.wait()


def _interp_gather(pt, di_flat, dw_splat):
    mesh = plsc.VectorSubcoreMesh(core_axis_name="c", subcore_axis_name="s")
    return pl.kernel(
        _sc_body,
        mesh=mesh,
        out_type=jax.ShapeDtypeStruct((B * Q, D1), jnp.float32),
        scratch_types=(
            [pltpu.VMEM((3 * SUBQ,), jnp.int32) for _ in range(2)]
            + [pltpu.VMEM((SUBQ, 48), jnp.float32)]
            + [pltpu.VMEM((3 * SUBQ, D1), jnp.float32) for _ in range(2)]
            + [pltpu.VMEM((SUBQ, D1), jnp.float32)]
            + [pltpu.SemaphoreType.DMA for _ in range(6)]
        ),
    )(pt, di_flat, dw_splat)


def _kernel_c1(qf_ref, w1_ref, b1_ref, z_ref, y1_ref, s1_ref, ss1_ref):
    b = pl.program_id(0)
    qt = pl.program_id(1)
    yq = lax.dot_general(w1_ref[:, :Cq], qf_ref[0], (((1,), (0,)), ((), ())),
                         preferred_element_type=jnp.float32)
    y1 = yq + b1_ref[...] + jnp.transpose(z_ref[...])
    y1_ref[0] = y1

    @pl.when(jnp.logical_and(b == 0, qt == 0))
    def _():
        s1_ref[...] = jnp.zeros_like(s1_ref)
        ss1_ref[...] = jnp.zeros_like(ss1_ref)

    s1_ref[...] += jnp.sum(y1, axis=1, keepdims=True)
    ss1_ref[...] += jnp.sum(y1 * y1, axis=1, keepdims=True)


def _kernel_b(y1_ref, s1_ref, ss1_ref, g1_ref, be1_ref, w2_ref, b2_ref,
              y2_ref, s2_ref, ss2_ref):
    b = pl.program_id(0)
    qt = pl.program_id(1)

    mean = s1_ref[...] * (1.0 / N_BN)
    var = ss1_ref[...] * (1.0 / N_BN) - mean * mean
    scale = g1_ref[...] * lax.rsqrt(var + 1e-5)
    shift = be1_ref[...] - mean * scale
    x1 = jnp.maximum(y1_ref[0] * scale + shift, 0.0)
    y2 = lax.dot_general(w2_ref[...], x1, (((1,), (0,)), ((), ())),
                         preferred_element_type=jnp.float32) + b2_ref[...]
    y2_ref[0] = y2

    @pl.when(jnp.logical_and(b == 0, qt == 0))
    def _():
        s2_ref[...] = jnp.zeros_like(s2_ref)
        ss2_ref[...] = jnp.zeros_like(ss2_ref)

    s2_ref[...] += jnp.sum(y2, axis=1, keepdims=True)
    ss2_ref[...] += jnp.sum(y2 * y2, axis=1, keepdims=True)


def _kernel_c(y2_ref, s2_ref, ss2_ref, g2_ref, be2_ref, out_ref):
    mean = s2_ref[...] * (1.0 / N_BN)
    var = ss2_ref[...] * (1.0 / N_BN) - mean * mean
    scale = g2_ref[...] * lax.rsqrt(var + 1e-5)
    shift = be2_ref[...] - mean * scale
    out_ref[0] = jnp.maximum(y2_ref[0] * scale + shift, 0.0)


def kernel(q_points, s_points, q_feats, s_feats, W1, b1, g1, be1,
           W2, b2, g2, be2):
    qpt = q_points.transpose(0, 2, 1)
    b1c = b1.reshape(D1, 1)
    g1c = g1.reshape(D1, 1)
    be1c = be1.reshape(D1, 1)
    b2c = b2.reshape(D2, 1)
    g2c = g2.reshape(D2, 1)
    be2c = be2.reshape(D2, 1)

    col = lambda d: pl.BlockSpec((d, 1), lambda b, q: (0, 0))

    di, dw = pl.pallas_call(
        _kernel_a1,
        grid=(B, NQT),
        in_specs=[
            pl.BlockSpec((1, QT, 3), lambda b, q: (b, q, 0)),
            pl.BlockSpec((1, 3, S), lambda b, q: (b, 0, 0)),
        ],
        out_specs=[
            pl.BlockSpec((1, QT, 3), lambda b, q: (b, q, 0)),
            pl.BlockSpec((QT, 48), lambda b, q: (b * NQT + q, 0)),
        ],
        out_shape=[
            jax.ShapeDtypeStruct((B, Q, 3), jnp.int32),
            jax.ShapeDtypeStruct((B * Q, 48), jnp.float32),
        ],
    )(qpt, s_points)

    pt = pl.pallas_call(
        _kernel_a0,
        grid=(B,),
        in_specs=[
            pl.BlockSpec((1, Cs, S), lambda b: (b, 0, 0)),
            pl.BlockSpec((D1, D0), lambda b: (0, 0)),
        ],
        out_specs=pl.BlockSpec((S, D1), lambda b: (b, 0)),
        out_shape=jax.ShapeDtypeStruct((B * S, D1), jnp.float32),
    )(s_feats, W1)

    z = _interp_gather(pt, di.reshape(-1), dw)

    y1, s1, ss1 = pl.pallas_call(
        _kernel_c1,
        grid=(B, NQT2),
        in_specs=[
            pl.BlockSpec((1, Cq, QT2), lambda b, q: (b, 0, q)),
            pl.BlockSpec((D1, D0), lambda b, q: (0, 0)),
            col(D1),
            pl.BlockSpec((QT2, D1), lambda b, q: (b * NQT2 + q, 0)),
        ],
        out_specs=[
            pl.BlockSpec((1, D1, QT2), lambda b, q: (b, 0, q)),
            col(D1),
            col(D1),
        ],
        out_shape=[
            jax.ShapeDtypeStruct((B, D1, Q), jnp.float32),
            jax.ShapeDtypeStruct((D1, 1), jnp.float32),
            jax.ShapeDtypeStruct((D1, 1), jnp.float32),
        ],
    )(q_feats, W1, b1c, z)

    y2, s2, ss2 = pl.pallas_call(
        _kernel_b,
        grid=(B, NQT2),
        in_specs=[
            pl.BlockSpec((1, D1, QT2), lambda b, q: (b, 0, q)),
            col(D1), col(D1), col(D1), col(D1),
            pl.BlockSpec((D2, D1), lambda b, q: (0, 0)),
            col(D2),
        ],
        out_specs=[
            pl.BlockSpec((1, D2, QT2), lambda b, q: (b, 0, q)),
            col(D2),
            col(D2),
        ],
        out_shape=[
            jax.ShapeDtypeStruct((B, D2, Q), jnp.float32),
            jax.ShapeDtypeStruct((D2, 1), jnp.float32),
            jax.ShapeDtypeStruct((D2, 1), jnp.float32),
        ],
    )(y1, s1, ss1, g1c, be1c, W2, b2c)

    out = pl.pallas_call(
        _kernel_c,
        grid=(B, NQT2),
        in_specs=[
            pl.BlockSpec((1, D2, QT2), lambda b, q: (b, 0, q)),
            col(D2), col(D2), col(D2), col(D2),
        ],
        out_specs=pl.BlockSpec((1, D2, QT2), lambda b, q: (b, 0, q)),
        out_shape=jax.ShapeDtypeStruct((B, D2, Q), jnp.float32),
    )(y2, s2, ss2, g2c, be2c)
    return out


# speedup vs baseline: 2.0530x; 1.0591x over previous
import functools

import jax
import jax.numpy as jnp
from jax import lax
from jax.experimental import pallas as pl
from jax.experimental.pallas import tpu as pltpu
from jax.experimental.pallas import tpu_sc as plsc

B, Q, S, Cq, Cs = 8, 4096, 1024, 128, 256
D0, D1, D2 = Cq + Cs, 256, 128
QT = 256
NQT = Q // QT
QT2 = 2048
NQT2 = Q // QT2
N_BN = B * Q

NW = 32
QPW = (B * Q) // NW
SUBQ = 64
NCH = QPW // SUBQ


def _kernel_a1(qpt_ref, sp_ref, di_ref, dw_ref):
    b = pl.program_id(0)

    qpt = qpt_ref[0]
    sp = sp_ref[0]
    qq = jnp.sum(qpt * qpt, axis=1, keepdims=True)
    ss = jnp.sum(sp * sp, axis=0, keepdims=True)
    qs = lax.dot_general(qpt, sp, (((1,), (0,)), ((), ())),
                         preferred_element_type=jnp.float32)
    d2 = qq + ss - 2.0 * qs

    inf = jnp.float32(jnp.inf)
    m1 = jnp.min(d2, axis=1, keepdims=True)
    m2 = jnp.min(jnp.where(d2 > m1, d2, inf), axis=1, keepdims=True)
    m3 = jnp.min(jnp.where(d2 > m2, d2, inf), axis=1, keepdims=True)

    iota = lax.broadcasted_iota(jnp.int32, (QT, S), 1)
    gbase = b * S
    iks = [jnp.min(jnp.where(d2 == m, iota, S), axis=1, keepdims=True) + gbase
           for m in (m1, m2, m3)]

    ws = [1.0 / (jnp.maximum(m, 0.0) + 1e-5) for m in (m1, m2, m3)]
    wsum = ws[0] + ws[1] + ws[2]
    ws = [w / wsum for w in ws]

    di_ref[0] = jnp.concatenate(iks, axis=1)
    dw_ref[...] = jnp.concatenate(
        [jnp.broadcast_to(w, (QT, 16)) for w in ws], axis=1)


def _kernel_a0(sf_ref, w1_ref, pt_ref):
    pt_ref[...] = lax.dot_general(
        sf_ref[0], w1_ref[:, Cq:], (((0,), (1,)), ((), ())),
        preferred_element_type=jnp.float32)


def _sc_body(pt_hbm, di_hbm, dw_hbm, z_hbm,
             idx0, idx1, w0_, r0, r1, o0,
             si0, si1, sw0, sg0, sg1, so0):
    c = lax.axis_index("c")
    s = lax.axis_index("s")
    wid = s * 2 + c
    n0 = wid * QPW
    idxb, rb = [idx0, idx1], [r0, r1]
    sib, sgb = [si0, si1], [sg0, sg1]

    def start_idx(ch, sl):
        row0 = n0 + ch * SUBQ
        return pltpu.async_copy(di_hbm.at[pl.ds(3 * row0, 3 * SUBQ)],
                                idxb[sl], sib[sl])

    def start_w(ch):
        row0 = n0 + ch * SUBQ
        return pltpu.async_copy(dw_hbm.at[pl.ds(row0, SUBQ)], w0_, sw0)

    def compute(sl):
        rows_v = rb[sl]

        def body(q, carry):
            base = 3 * q
            wv0 = w0_[q, pl.ds(0, 16)]
            wv1 = w0_[q, pl.ds(16, 16)]
            wv2 = w0_[q, pl.ds(32, 16)]
            for j in range(D1 // 16):
                slc = pl.ds(16 * j, 16)
                o0[q, slc] = (
                    wv0 * rows_v[base, slc] + wv1 * rows_v[base + 1, slc]
                    + wv2 * rows_v[base + 2, slc])
            return carry

        lax.fori_loop(0, SUBQ, body, 0)

    hidx = [None, None]
    hg = [None, None]
    hout = None
    hidx[0] = start_idx(0, 0)
    hidx[1] = start_idx(1, 1)
    hw = start_w(0)
    hidx[0].wait()
    hg[0] = pltpu.async_copy(pt_hbm.at[idxb[0]], rb[0], sgb[0])
    for ch in range(NCH):
        sl = ch % 2
        if ch + 1 < NCH:
            hidx[1 - sl].wait()
            hg[1 - sl] = pltpu.async_copy(pt_hbm.at[idxb[1 - sl]],
                                          rb[1 - sl], sgb[1 - sl])
        hg[sl].wait()
        hw.wait()
        if hout is not None:
            hout.wait()
        compute(sl)
        hout = pltpu.async_copy(o0, z_hbm.at[pl.ds(n0 + ch * SUBQ, SUBQ)],
                                so0)
        if ch + 2 < NCH:
            hidx[sl] = start_idx(ch + 2, sl)
        if ch + 1 < NCH:
            hw = start_w(ch + 1)
    hout.wait()


def _interp_gather(pt, di_flat, dw_splat):
    mesh = plsc.VectorSubcoreMesh(core_axis_name="c", subcore_axis_name="s")
    return pl.kernel(
        _sc_body,
        mesh=mesh,
        out_type=jax.ShapeDtypeStruct((B * Q, D1), jnp.float32),
        scratch_types=(
            [pltpu.VMEM((3 * SUBQ,), jnp.int32) for _ in range(2)]
            + [pltpu.VMEM((SUBQ, 48), jnp.float32)]
            + [pltpu.VMEM((3 * SUBQ, D1), jnp.float32) for _ in range(2)]
            + [pltpu.VMEM((SUBQ, D1), jnp.float32)]
            + [pltpu.SemaphoreType.DMA for _ in range(6)]
        ),
    )(pt, di_flat, dw_splat)


def _kernel_c1(qf_ref, w1_ref, b1_ref, z_ref, y1_ref, s1_ref, ss1_ref):
    b = pl.program_id(0)
    qt = pl.program_id(1)
    yq = lax.dot_general(w1_ref[:, :Cq], qf_ref[0], (((1,), (0,)), ((), ())),
                         preferred_element_type=jnp.float32)
    y1 = yq + b1_ref[...] + jnp.transpose(z_ref[...])
    y1_ref[0] = y1

    @pl.when(jnp.logical_and(b == 0, qt == 0))
    def _():
        s1_ref[...] = jnp.zeros_like(s1_ref)
        ss1_ref[...] = jnp.zeros_like(ss1_ref)

    s1_ref[...] += jnp.sum(y1, axis=1, keepdims=True)
    ss1_ref[...] += jnp.sum(y1 * y1, axis=1, keepdims=True)


def _kernel_b(y1_ref, s1_ref, ss1_ref, g1_ref, be1_ref, w2_ref, b2_ref,
              y2_ref, s2_ref, ss2_ref):
    b = pl.program_id(0)
    qt = pl.program_id(1)

    mean = s1_ref[...] * (1.0 / N_BN)
    var = ss1_ref[...] * (1.0 / N_BN) - mean * mean
    scale = g1_ref[...] * lax.rsqrt(var + 1e-5)
    shift = be1_ref[...] - mean * scale
    x1 = jnp.maximum(y1_ref[0] * scale + shift, 0.0)
    y2 = lax.dot_general(w2_ref[...], x1, (((1,), (0,)), ((), ())),
                         preferred_element_type=jnp.float32) + b2_ref[...]
    y2_ref[0] = y2

    @pl.when(jnp.logical_and(b == 0, qt == 0))
    def _():
        s2_ref[...] = jnp.zeros_like(s2_ref)
        ss2_ref[...] = jnp.zeros_like(ss2_ref)

    s2_ref[...] += jnp.sum(y2, axis=1, keepdims=True)
    ss2_ref[...] += jnp.sum(y2 * y2, axis=1, keepdims=True)


def _kernel_c(y2_ref, s2_ref, ss2_ref, g2_ref, be2_ref, out_ref):
    mean = s2_ref[...] * (1.0 / N_BN)
    var = ss2_ref[...] * (1.0 / N_BN) - mean * mean
    scale = g2_ref[...] * lax.rsqrt(var + 1e-5)
    shift = be2_ref[...] - mean * scale
    out_ref[0] = jnp.maximum(y2_ref[0] * scale + shift, 0.0)


def kernel(q_points, s_points, q_feats, s_feats, W1, b1, g1, be1,
           W2, b2, g2, be2):
    qpt = q_points.transpose(0, 2, 1)
    b1c = b1.reshape(D1, 1)
    g1c = g1.reshape(D1, 1)
    be1c = be1.reshape(D1, 1)
    b2c = b2.reshape(D2, 1)
    g2c = g2.reshape(D2, 1)
    be2c = be2.reshape(D2, 1)

    col = lambda d: pl.BlockSpec((d, 1), lambda b, q: (0, 0))

    di, dw = pl.pallas_call(
        _kernel_a1,
        grid=(B, NQT),
        in_specs=[
            pl.BlockSpec((1, QT, 3), lambda b, q: (b, q, 0)),
            pl.BlockSpec((1, 3, S), lambda b, q: (b, 0, 0)),
        ],
        out_specs=[
            pl.BlockSpec((1, QT, 3), lambda b, q: (b, q, 0)),
            pl.BlockSpec((QT, 48), lambda b, q: (b * NQT + q, 0)),
        ],
        out_shape=[
            jax.ShapeDtypeStruct((B, Q, 3), jnp.int32),
            jax.ShapeDtypeStruct((B * Q, 48), jnp.float32),
        ],
    )(qpt, s_points)

    pt = pl.pallas_call(
        _kernel_a0,
        grid=(B,),
        in_specs=[
            pl.BlockSpec((1, Cs, S), lambda b: (b, 0, 0)),
            pl.BlockSpec((D1, D0), lambda b: (0, 0)),
        ],
        out_specs=pl.BlockSpec((S, D1), lambda b: (b, 0)),
        out_shape=jax.ShapeDtypeStruct((B * S, D1), jnp.float32),
    )(s_feats, W1)

    z = _interp_gather(pt, di.reshape(-1), dw)

    y1, s1, ss1 = pl.pallas_call(
        _kernel_c1,
        grid=(B, NQT2),
        in_specs=[
            pl.BlockSpec((1, Cq, QT2), lambda b, q: (b, 0, q)),
            pl.BlockSpec((D1, D0), lambda b, q: (0, 0)),
            col(D1),
            pl.BlockSpec((QT2, D1), lambda b, q: (b * NQT2 + q, 0)),
        ],
        out_specs=[
            pl.BlockSpec((1, D1, QT2), lambda b, q: (b, 0, q)),
            col(D1),
            col(D1),
        ],
        out_shape=[
            jax.ShapeDtypeStruct((B, D1, Q), jnp.float32),
            jax.ShapeDtypeStruct((D1, 1), jnp.float32),
            jax.ShapeDtypeStruct((D1, 1), jnp.float32),
        ],
    )(q_feats, W1, b1c, z)

    y2, s2, ss2 = pl.pallas_call(
        _kernel_b,
        grid=(B, NQT2),
        in_specs=[
            pl.BlockSpec((1, D1, QT2), lambda b, q: (b, 0, q)),
            col(D1), col(D1), col(D1), col(D1),
            pl.BlockSpec((D2, D1), lambda b, q: (0, 0)),
            col(D2),
        ],
        out_specs=[
            pl.BlockSpec((1, D2, QT2), lambda b, q: (b, 0, q)),
            col(D2),
            col(D2),
        ],
        out_shape=[
            jax.ShapeDtypeStruct((B, D2, Q), jnp.float32),
            jax.ShapeDtypeStruct((D2, 1), jnp.float32),
            jax.ShapeDtypeStruct((D2, 1), jnp.float32),
        ],
    )(y1, s1, ss1, g1c, be1c, W2, b2c)

    out = pl.pallas_call(
        _kernel_c,
        grid=(B, NQT2),
        in_specs=[
            pl.BlockSpec((1, D2, QT2), lambda b, q: (b, 0, q)),
            col(D2), col(D2), col(D2), col(D2),
        ],
        out_specs=pl.BlockSpec((1, D2, QT2), lambda b, q: (b, 0, q)),
        out_shape=jax.ShapeDtypeStruct((B, D2, Q), jnp.float32),
    )(y2, s2, ss2, g2c, be2c)
    return out


# speedup vs baseline: 2.1163x; 1.0308x over previous
import functools

import jax
import jax.numpy as jnp
from jax import lax
from jax.experimental import pallas as pl
from jax.experimental.pallas import tpu as pltpu
from jax.experimental.pallas import tpu_sc as plsc

B, Q, S, Cq, Cs = 8, 4096, 1024, 128, 256
D0, D1, D2 = Cq + Cs, 256, 128
QT = 256
NQT = Q // QT
QT2 = 4096
NQT2 = Q // QT2
N_BN = B * Q

NW = 32
QPW = (B * Q) // NW
SUBQ = 64
NCH = QPW // SUBQ


def _kernel_a1(qpt_ref, sp_ref, di_ref, dw_ref):
    b = pl.program_id(0)

    qpt = qpt_ref[0]
    sp = sp_ref[0]
    qq = jnp.sum(qpt * qpt, axis=1, keepdims=True)
    ss = jnp.sum(sp * sp, axis=0, keepdims=True)
    qs = lax.dot_general(qpt, sp, (((1,), (0,)), ((), ())),
                         preferred_element_type=jnp.float32)
    d2 = qq + ss - 2.0 * qs

    inf = jnp.float32(jnp.inf)
    m1 = jnp.min(d2, axis=1, keepdims=True)
    m2 = jnp.min(jnp.where(d2 > m1, d2, inf), axis=1, keepdims=True)
    m3 = jnp.min(jnp.where(d2 > m2, d2, inf), axis=1, keepdims=True)

    iota = lax.broadcasted_iota(jnp.int32, (QT, S), 1)
    gbase = b * S
    iks = [jnp.min(jnp.where(d2 == m, iota, S), axis=1, keepdims=True) + gbase
           for m in (m1, m2, m3)]

    ws = [1.0 / (jnp.maximum(m, 0.0) + 1e-5) for m in (m1, m2, m3)]
    wsum = ws[0] + ws[1] + ws[2]
    ws = [w / wsum for w in ws]

    di_ref[0] = jnp.concatenate(iks, axis=1)
    dw_ref[...] = jnp.concatenate(
        [jnp.broadcast_to(w, (QT, 16)) for w in ws], axis=1)


def _kernel_a0(sf_ref, w1_ref, pt_ref):
    pt_ref[...] = lax.dot_general(
        sf_ref[0], w1_ref[:, Cq:], (((0,), (1,)), ((), ())),
        preferred_element_type=jnp.float32)


def _sc_body(pt_hbm, di_hbm, dw_hbm, z_hbm,
             idx0, idx1, w0_, r0, r1, o0,
             si0, si1, sw0, sg0, sg1, so0):
    c = lax.axis_index("c")
    s = lax.axis_index("s")
    wid = s * 2 + c
    n0 = wid * QPW
    idxb, rb = [idx0, idx1], [r0, r1]
    sib, sgb = [si0, si1], [sg0, sg1]

    def start_idx(ch, sl):
        row0 = n0 + ch * SUBQ
        return pltpu.async_copy(di_hbm.at[pl.ds(3 * row0, 3 * SUBQ)],
                                idxb[sl], sib[sl])

    def start_w(ch):
        row0 = n0 + ch * SUBQ
        return pltpu.async_copy(dw_hbm.at[pl.ds(row0, SUBQ)], w0_, sw0)

    def compute(sl):
        rows_v = rb[sl]

        def body(q, carry):
            base = 3 * q
            wv0 = w0_[q, pl.ds(0, 16)]
            wv1 = w0_[q, pl.ds(16, 16)]
            wv2 = w0_[q, pl.ds(32, 16)]
            for j in range(D1 // 16):
                slc = pl.ds(16 * j, 16)
                o0[q, slc] = (
                    wv0 * rows_v[base, slc] + wv1 * rows_v[base + 1, slc]
                    + wv2 * rows_v[base + 2, slc])
            return carry

        lax.fori_loop(0, SUBQ, body, 0)

    hidx = [None, None]
    hg = [None, None]
    hout = None
    hidx[0] = start_idx(0, 0)
    hidx[1] = start_idx(1, 1)
    hw = start_w(0)
    hidx[0].wait()
    hg[0] = pltpu.async_copy(pt_hbm.at[idxb[0]], rb[0], sgb[0])
    for ch in range(NCH):
        sl = ch % 2
        if ch + 1 < NCH:
            hidx[1 - sl].wait()
            hg[1 - sl] = pltpu.async_copy(pt_hbm.at[idxb[1 - sl]],
                                          rb[1 - sl], sgb[1 - sl])
        hg[sl].wait()
        hw.wait()
        if hout is not None:
            hout.wait()
        compute(sl)
        hout = pltpu.async_copy(o0, z_hbm.at[pl.ds(n0 + ch * SUBQ, SUBQ)],
                                so0)
        if ch + 2 < NCH:
            hidx[sl] = start_idx(ch + 2, sl)
        if ch + 1 < NCH:
            hw = start_w(ch + 1)
    hout.wait()


def _interp_gather(pt, di_flat, dw_splat):
    mesh = plsc.VectorSubcoreMesh(core_axis_name="c", subcore_axis_name="s")
    return pl.kernel(
        _sc_body,
        mesh=mesh,
        out_type=jax.ShapeDtypeStruct((B * Q, D1), jnp.float32),
        scratch_types=(
            [pltpu.VMEM((3 * SUBQ,), jnp.int32) for _ in range(2)]
            + [pltpu.VMEM((SUBQ, 48), jnp.float32)]
            + [pltpu.VMEM((3 * SUBQ, D1), jnp.float32) for _ in range(2)]
            + [pltpu.VMEM((SUBQ, D1), jnp.float32)]
            + [pltpu.SemaphoreType.DMA for _ in range(6)]
        ),
    )(pt, di_flat, dw_splat)


def _kernel_c1(qf_ref, w1_ref, b1_ref, z_ref, y1_ref, s1_ref, ss1_ref):
    b = pl.program_id(0)
    qt = pl.program_id(1)
    yq = lax.dot_general(w1_ref[:, :Cq], qf_ref[0], (((1,), (0,)), ((), ())),
                         preferred_element_type=jnp.float32)
    y1 = yq + b1_ref[...] + jnp.transpose(z_ref[...])
    y1_ref[0] = y1

    @pl.when(jnp.logical_and(b == 0, qt == 0))
    def _():
        s1_ref[...] = jnp.zeros_like(s1_ref)
        ss1_ref[...] = jnp.zeros_like(ss1_ref)

    s1_ref[...] += jnp.sum(y1, axis=1, keepdims=True)
    ss1_ref[...] += jnp.sum(y1 * y1, axis=1, keepdims=True)


def _kernel_b(y1_ref, s1_ref, ss1_ref, g1_ref, be1_ref, w2_ref, b2_ref,
              y2_ref, s2_ref, ss2_ref):
    b = pl.program_id(0)
    qt = pl.program_id(1)

    mean = s1_ref[...] * (1.0 / N_BN)
    var = ss1_ref[...] * (1.0 / N_BN) - mean * mean
    scale = g1_ref[...] * lax.rsqrt(var + 1e-5)
    shift = be1_ref[...] - mean * scale
    x1 = jnp.maximum(y1_ref[0] * scale + shift, 0.0)
    y2 = lax.dot_general(w2_ref[...], x1, (((1,), (0,)), ((), ())),
                         preferred_element_type=jnp.float32) + b2_ref[...]
    y2_ref[0] = y2

    @pl.when(jnp.logical_and(b == 0, qt == 0))
    def _():
        s2_ref[...] = jnp.zeros_like(s2_ref)
        ss2_ref[...] = jnp.zeros_like(ss2_ref)

    s2_ref[...] += jnp.sum(y2, axis=1, keepdims=True)
    ss2_ref[...] += jnp.sum(y2 * y2, axis=1, keepdims=True)


def _kernel_c(y2_ref, s2_ref, ss2_ref, g2_ref, be2_ref, out_ref):
    mean = s2_ref[...] * (1.0 / N_BN)
    var = ss2_ref[...] * (1.0 / N_BN) - mean * mean
    scale = g2_ref[...] * lax.rsqrt(var + 1e-5)
    shift = be2_ref[...] - mean * scale
    out_ref[0] = jnp.maximum(y2_ref[0] * scale + shift, 0.0)


def kernel(q_points, s_points, q_feats, s_feats, W1, b1, g1, be1,
           W2, b2, g2, be2):
    qpt = q_points.transpose(0, 2, 1)
    b1c = b1.reshape(D1, 1)
    g1c = g1.reshape(D1, 1)
    be1c = be1.reshape(D1, 1)
    b2c = b2.reshape(D2, 1)
    g2c = g2.reshape(D2, 1)
    be2c = be2.reshape(D2, 1)

    col = lambda d: pl.BlockSpec((d, 1), lambda b, q: (0, 0))

    di, dw = pl.pallas_call(
        _kernel_a1,
        grid=(B, NQT),
        in_specs=[
            pl.BlockSpec((1, QT, 3), lambda b, q: (b, q, 0)),
            pl.BlockSpec((1, 3, S), lambda b, q: (b, 0, 0)),
        ],
        out_specs=[
            pl.BlockSpec((1, QT, 3), lambda b, q: (b, q, 0)),
            pl.BlockSpec((QT, 48), lambda b, q: (b * NQT + q, 0)),
        ],
        out_shape=[
            jax.ShapeDtypeStruct((B, Q, 3), jnp.int32),
            jax.ShapeDtypeStruct((B * Q, 48), jnp.float32),
        ],
    )(qpt, s_points)

    pt = pl.pallas_call(
        _kernel_a0,
        grid=(B,),
        in_specs=[
            pl.BlockSpec((1, Cs, S), lambda b: (b, 0, 0)),
            pl.BlockSpec((D1, D0), lambda b: (0, 0)),
        ],
        out_specs=pl.BlockSpec((S, D1), lambda b: (b, 0)),
        out_shape=jax.ShapeDtypeStruct((B * S, D1), jnp.float32),
    )(s_feats, W1)

    z = _interp_gather(pt, di.reshape(-1), dw)

    y1, s1, ss1 = pl.pallas_call(
        _kernel_c1,
        grid=(B, NQT2),
        in_specs=[
            pl.BlockSpec((1, Cq, QT2), lambda b, q: (b, 0, q)),
            pl.BlockSpec((D1, D0), lambda b, q: (0, 0)),
            col(D1),
            pl.BlockSpec((QT2, D1), lambda b, q: (b * NQT2 + q, 0)),
        ],
        out_specs=[
            pl.BlockSpec((1, D1, QT2), lambda b, q: (b, 0, q)),
            col(D1),
            col(D1),
        ],
        out_shape=[
            jax.ShapeDtypeStruct((B, D1, Q), jnp.float32),
            jax.ShapeDtypeStruct((D1, 1), jnp.float32),
            jax.ShapeDtypeStruct((D1, 1), jnp.float32),
        ],
    )(q_feats, W1, b1c, z)

    y2, s2, ss2 = pl.pallas_call(
        _kernel_b,
        grid=(B, NQT2),
        in_specs=[
            pl.BlockSpec((1, D1, QT2), lambda b, q: (b, 0, q)),
            col(D1), col(D1), col(D1), col(D1),
            pl.BlockSpec((D2, D1), lambda b, q: (0, 0)),
            col(D2),
        ],
        out_specs=[
            pl.BlockSpec((1, D2, QT2), lambda b, q: (b, 0, q)),
            col(D2),
            col(D2),
        ],
        out_shape=[
            jax.ShapeDtypeStruct((B, D2, Q), jnp.float32),
            jax.ShapeDtypeStruct((D2, 1), jnp.float32),
            jax.ShapeDtypeStruct((D2, 1), jnp.float32),
        ],
    )(y1, s1, ss1, g1c, be1c, W2, b2c)

    out = pl.pallas_call(
        _kernel_c,
        grid=(B, NQT2),
        in_specs=[
            pl.BlockSpec((1, D2, QT2), lambda b, q: (b, 0, q)),
            col(D2), col(D2), col(D2), col(D2),
        ],
        out_specs=pl.BlockSpec((1, D2, QT2), lambda b, q: (b, 0, q)),
        out_shape=jax.ShapeDtypeStruct((B, D2, Q), jnp.float32),
    )(y2, s2, ss2, g2c, be2c)
    return out


# speedup vs baseline: 2.2236x; 1.0507x over previous
import functools

import jax
import jax.numpy as jnp
from jax import lax
from jax.experimental import pallas as pl
from jax.experimental.pallas import tpu as pltpu
from jax.experimental.pallas import tpu_sc as plsc

B, Q, S, Cq, Cs = 8, 4096, 1024, 128, 256
D0, D1, D2 = Cq + Cs, 256, 128
QT = 512
NQT = Q // QT
QT2 = 4096
NQT2 = Q // QT2
N_BN = B * Q

NW = 32
QPW = (B * Q) // NW
SUBQ = 64
NCH = QPW // SUBQ


def _kernel_a1(qpt_ref, sp_ref, di_ref, dw_ref):
    b = pl.program_id(0)

    qpt = qpt_ref[0]
    sp = sp_ref[0]
    qq = jnp.sum(qpt * qpt, axis=1, keepdims=True)
    ss = jnp.sum(sp * sp, axis=0, keepdims=True)
    qs = lax.dot_general(qpt, sp, (((1,), (0,)), ((), ())),
                         preferred_element_type=jnp.float32)
    d2 = qq + ss - 2.0 * qs

    inf = jnp.float32(jnp.inf)
    m1 = jnp.min(d2, axis=1, keepdims=True)
    m2 = jnp.min(jnp.where(d2 > m1, d2, inf), axis=1, keepdims=True)
    m3 = jnp.min(jnp.where(d2 > m2, d2, inf), axis=1, keepdims=True)

    iota = lax.broadcasted_iota(jnp.int32, (QT, S), 1)
    gbase = b * S
    iks = [jnp.min(jnp.where(d2 == m, iota, S), axis=1, keepdims=True) + gbase
           for m in (m1, m2, m3)]

    ws = [1.0 / (jnp.maximum(m, 0.0) + 1e-5) for m in (m1, m2, m3)]
    wsum = ws[0] + ws[1] + ws[2]
    ws = [w / wsum for w in ws]

    di_ref[0] = jnp.concatenate(iks, axis=1)
    dw_ref[...] = jnp.concatenate(
        [jnp.broadcast_to(w, (QT, 16)) for w in ws], axis=1)


def _kernel_a0(sf_ref, w1_ref, pt_ref):
    pt_ref[...] = lax.dot_general(
        sf_ref[0], w1_ref[:, Cq:], (((0,), (1,)), ((), ())),
        preferred_element_type=jnp.float32)


def _sc_body(pt_hbm, di_hbm, dw_hbm, z_hbm,
             idx0, idx1, w0_, r0, r1, o0,
             si0, si1, sw0, sg0, sg1, so0):
    c = lax.axis_index("c")
    s = lax.axis_index("s")
    wid = s * 2 + c
    n0 = wid * QPW
    idxb, rb = [idx0, idx1], [r0, r1]
    sib, sgb = [si0, si1], [sg0, sg1]

    def start_idx(ch, sl):
        row0 = n0 + ch * SUBQ
        return pltpu.async_copy(di_hbm.at[pl.ds(3 * row0, 3 * SUBQ)],
                                idxb[sl], sib[sl])

    def start_w(ch):
        row0 = n0 + ch * SUBQ
        return pltpu.async_copy(dw_hbm.at[pl.ds(row0, SUBQ)], w0_, sw0)

    def compute(sl):
        rows_v = rb[sl]

        def body(q, carry):
            base = 3 * q
            wv0 = w0_[q, pl.ds(0, 16)]
            wv1 = w0_[q, pl.ds(16, 16)]
            wv2 = w0_[q, pl.ds(32, 16)]
            for j in range(D1 // 16):
                slc = pl.ds(16 * j, 16)
                o0[q, slc] = (
                    wv0 * rows_v[base, slc] + wv1 * rows_v[base + 1, slc]
                    + wv2 * rows_v[base + 2, slc])
            return carry

        lax.fori_loop(0, SUBQ, body, 0)

    hidx = [None, None]
    hg = [None, None]
    hout = None
    hidx[0] = start_idx(0, 0)
    hidx[1] = start_idx(1, 1)
    hw = start_w(0)
    hidx[0].wait()
    hg[0] = pltpu.async_copy(pt_hbm.at[idxb[0]], rb[0], sgb[0])
    for ch in range(NCH):
        sl = ch % 2
        if ch + 1 < NCH:
            hidx[1 - sl].wait()
            hg[1 - sl] = pltpu.async_copy(pt_hbm.at[idxb[1 - sl]],
                                          rb[1 - sl], sgb[1 - sl])
        hg[sl].wait()
        hw.wait()
        if hout is not None:
            hout.wait()
        compute(sl)
        hout = pltpu.async_copy(o0, z_hbm.at[pl.ds(n0 + ch * SUBQ, SUBQ)],
                                so0)
        if ch + 2 < NCH:
            hidx[sl] = start_idx(ch + 2, sl)
        if ch + 1 < NCH:
            hw = start_w(ch + 1)
    hout.wait()


def _interp_gather(pt, di_flat, dw_splat):
    mesh = plsc.VectorSubcoreMesh(core_axis_name="c", subcore_axis_name="s")
    return pl.kernel(
        _sc_body,
        mesh=mesh,
        out_type=jax.ShapeDtypeStruct((B * Q, D1), jnp.float32),
        scratch_types=(
            [pltpu.VMEM((3 * SUBQ,), jnp.int32) for _ in range(2)]
            + [pltpu.VMEM((SUBQ, 48), jnp.float32)]
            + [pltpu.VMEM((3 * SUBQ, D1), jnp.float32) for _ in range(2)]
            + [pltpu.VMEM((SUBQ, D1), jnp.float32)]
            + [pltpu.SemaphoreType.DMA for _ in range(6)]
        ),
    )(pt, di_flat, dw_splat)


def _kernel_c1(qf_ref, w1_ref, b1_ref, z_ref, y1_ref, s1_ref, ss1_ref):
    b = pl.program_id(0)
    qt = pl.program_id(1)
    yq = lax.dot_general(w1_ref[:, :Cq], qf_ref[0], (((1,), (0,)), ((), ())),
                         preferred_element_type=jnp.float32)
    y1 = yq + b1_ref[...] + jnp.transpose(z_ref[...])
    y1_ref[0] = y1

    @pl.when(jnp.logical_and(b == 0, qt == 0))
    def _():
        s1_ref[...] = jnp.zeros_like(s1_ref)
        ss1_ref[...] = jnp.zeros_like(ss1_ref)

    s1_ref[...] += jnp.sum(y1, axis=1, keepdims=True)
    ss1_ref[...] += jnp.sum(y1 * y1, axis=1, keepdims=True)


def _kernel_b(y1_ref, s1_ref, ss1_ref, g1_ref, be1_ref, w2_ref, b2_ref,
              y2_ref, s2_ref, ss2_ref):
    b = pl.program_id(0)
    qt = pl.program_id(1)

    mean = s1_ref[...] * (1.0 / N_BN)
    var = ss1_ref[...] * (1.0 / N_BN) - mean * mean
    scale = g1_ref[...] * lax.rsqrt(var + 1e-5)
    shift = be1_ref[...] - mean * scale
    x1 = jnp.maximum(y1_ref[0] * scale + shift, 0.0)
    y2 = lax.dot_general(w2_ref[...], x1, (((1,), (0,)), ((), ())),
                         preferred_element_type=jnp.float32) + b2_ref[...]
    y2_ref[0] = y2

    @pl.when(jnp.logical_and(b == 0, qt == 0))
    def _():
        s2_ref[...] = jnp.zeros_like(s2_ref)
        ss2_ref[...] = jnp.zeros_like(ss2_ref)

    s2_ref[...] += jnp.sum(y2, axis=1, keepdims=True)
    ss2_ref[...] += jnp.sum(y2 * y2, axis=1, keepdims=True)


def _kernel_c(y2_ref, s2_ref, ss2_ref, g2_ref, be2_ref, out_ref):
    mean = s2_ref[...] * (1.0 / N_BN)
    var = ss2_ref[...] * (1.0 / N_BN) - mean * mean
    scale = g2_ref[...] * lax.rsqrt(var + 1e-5)
    shift = be2_ref[...] - mean * scale
    out_ref[0] = jnp.maximum(y2_ref[0] * scale + shift, 0.0)


def kernel(q_points, s_points, q_feats, s_feats, W1, b1, g1, be1,
           W2, b2, g2, be2):
    qpt = q_points.transpose(0, 2, 1)
    b1c = b1.reshape(D1, 1)
    g1c = g1.reshape(D1, 1)
    be1c = be1.reshape(D1, 1)
    b2c = b2.reshape(D2, 1)
    g2c = g2.reshape(D2, 1)
    be2c = be2.reshape(D2, 1)

    col = lambda d: pl.BlockSpec((d, 1), lambda b, q: (0, 0))

    di, dw = pl.pallas_call(
        _kernel_a1,
        grid=(B, NQT),
        in_specs=[
            pl.BlockSpec((1, QT, 3), lambda b, q: (b, q, 0)),
            pl.BlockSpec((1, 3, S), lambda b, q: (b, 0, 0)),
        ],
        out_specs=[
            pl.BlockSpec((1, QT, 3), lambda b, q: (b, q, 0)),
            pl.BlockSpec((QT, 48), lambda b, q: (b * NQT + q, 0)),
        ],
        out_shape=[
            jax.ShapeDtypeStruct((B, Q, 3), jnp.int32),
            jax.ShapeDtypeStruct((B * Q, 48), jnp.float32),
        ],
    )(qpt, s_points)

    pt = pl.pallas_call(
        _kernel_a0,
        grid=(B,),
        in_specs=[
            pl.BlockSpec((1, Cs, S), lambda b: (b, 0, 0)),
            pl.BlockSpec((D1, D0), lambda b: (0, 0)),
        ],
        out_specs=pl.BlockSpec((S, D1), lambda b: (b, 0)),
        out_shape=jax.ShapeDtypeStruct((B * S, D1), jnp.float32),
    )(s_feats, W1)

    z = _interp_gather(pt, di.reshape(-1), dw)

    y1, s1, ss1 = pl.pallas_call(
        _kernel_c1,
        grid=(B, NQT2),
        in_specs=[
            pl.BlockSpec((1, Cq, QT2), lambda b, q: (b, 0, q)),
            pl.BlockSpec((D1, D0), lambda b, q: (0, 0)),
            col(D1),
            pl.BlockSpec((QT2, D1), lambda b, q: (b * NQT2 + q, 0)),
        ],
        out_specs=[
            pl.BlockSpec((1, D1, QT2), lambda b, q: (b, 0, q)),
            col(D1),
            col(D1),
        ],
        out_shape=[
            jax.ShapeDtypeStruct((B, D1, Q), jnp.float32),
            jax.ShapeDtypeStruct((D1, 1), jnp.float32),
            jax.ShapeDtypeStruct((D1, 1), jnp.float32),
        ],
    )(q_feats, W1, b1c, z)

    y2, s2, ss2 = pl.pallas_call(
        _kernel_b,
        grid=(B, NQT2),
        in_specs=[
            pl.BlockSpec((1, D1, QT2), lambda b, q: (b, 0, q)),
            col(D1), col(D1), col(D1), col(D1),
            pl.BlockSpec((D2, D1), lambda b, q: (0, 0)),
            col(D2),
        ],
        out_specs=[
            pl.BlockSpec((1, D2, QT2), lambda b, q: (b, 0, q)),
            col(D2),
            col(D2),
        ],
        out_shape=[
            jax.ShapeDtypeStruct((B, D2, Q), jnp.float32),
            jax.ShapeDtypeStruct((D2, 1), jnp.float32),
            jax.ShapeDtypeStruct((D2, 1), jnp.float32),
        ],
    )(y1, s1, ss1, g1c, be1c, W2, b2c)

    out = pl.pallas_call(
        _kernel_c,
        grid=(B, NQT2),
        in_specs=[
            pl.BlockSpec((1, D2, QT2), lambda b, q: (b, 0, q)),
            col(D2), col(D2), col(D2), col(D2),
        ],
        out_specs=pl.BlockSpec((1, D2, QT2), lambda b, q: (b, 0, q)),
        out_shape=jax.ShapeDtypeStruct((B, D2, Q), jnp.float32),
    )(y2, s2, ss2, g2c, be2c)
    return out


# speedup vs baseline: 2.2349x; 1.0051x over previous
import functools

import jax
import jax.numpy as jnp
from jax import lax
from jax.experimental import pallas as pl
from jax.experimental.pallas import tpu as pltpu
from jax.experimental.pallas import tpu_sc as plsc

B, Q, S, Cq, Cs = 8, 4096, 1024, 128, 256
D0, D1, D2 = Cq + Cs, 256, 128
QT = 1024
NQT = Q // QT
QT2 = 4096
NQT2 = Q // QT2
N_BN = B * Q

NW = 32
QPW = (B * Q) // NW
SUBQ = 64
NCH = QPW // SUBQ


def _kernel_a1(qpt_ref, sp_ref, di_ref, dw_ref):
    b = pl.program_id(0)

    qpt = qpt_ref[0]
    sp = sp_ref[0]
    qq = jnp.sum(qpt * qpt, axis=1, keepdims=True)
    ss = jnp.sum(sp * sp, axis=0, keepdims=True)
    qs = lax.dot_general(qpt, sp, (((1,), (0,)), ((), ())),
                         preferred_element_type=jnp.float32)
    d2 = qq + ss - 2.0 * qs

    inf = jnp.float32(jnp.inf)
    m1 = jnp.min(d2, axis=1, keepdims=True)
    m2 = jnp.min(jnp.where(d2 > m1, d2, inf), axis=1, keepdims=True)
    m3 = jnp.min(jnp.where(d2 > m2, d2, inf), axis=1, keepdims=True)

    iota = lax.broadcasted_iota(jnp.int32, (QT, S), 1)
    gbase = b * S
    iks = [jnp.min(jnp.where(d2 == m, iota, S), axis=1, keepdims=True) + gbase
           for m in (m1, m2, m3)]

    ws = [1.0 / (jnp.maximum(m, 0.0) + 1e-5) for m in (m1, m2, m3)]
    wsum = ws[0] + ws[1] + ws[2]
    ws = [w / wsum for w in ws]

    di_ref[0] = jnp.concatenate(iks, axis=1)
    dw_ref[...] = jnp.concatenate(
        [jnp.broadcast_to(w, (QT, 16)) for w in ws], axis=1)


def _kernel_a0(sf_ref, w1_ref, pt_ref):
    pt_ref[...] = lax.dot_general(
        sf_ref[0], w1_ref[:, Cq:], (((0,), (1,)), ((), ())),
        preferred_element_type=jnp.float32)


def _sc_body(pt_hbm, di_hbm, dw_hbm, z_hbm,
             idx0, idx1, w0_, r0, r1, o0,
             si0, si1, sw0, sg0, sg1, so0):
    c = lax.axis_index("c")
    s = lax.axis_index("s")
    wid = s * 2 + c
    n0 = wid * QPW
    idxb, rb = [idx0, idx1], [r0, r1]
    sib, sgb = [si0, si1], [sg0, sg1]

    def start_idx(ch, sl):
        row0 = n0 + ch * SUBQ
        return pltpu.async_copy(di_hbm.at[pl.ds(3 * row0, 3 * SUBQ)],
                                idxb[sl], sib[sl])

    def start_w(ch):
        row0 = n0 + ch * SUBQ
        return pltpu.async_copy(dw_hbm.at[pl.ds(row0, SUBQ)], w0_, sw0)

    def compute(sl):
        rows_v = rb[sl]

        def body(q, carry):
            base = 3 * q
            wv0 = w0_[q, pl.ds(0, 16)]
            wv1 = w0_[q, pl.ds(16, 16)]
            wv2 = w0_[q, pl.ds(32, 16)]
            for j in range(D1 // 16):
                slc = pl.ds(16 * j, 16)
                o0[q, slc] = (
                    wv0 * rows_v[base, slc] + wv1 * rows_v[base + 1, slc]
                    + wv2 * rows_v[base + 2, slc])
            return carry

        lax.fori_loop(0, SUBQ, body, 0)

    hidx = [None, None]
    hg = [None, None]
    hout = None
    hidx[0] = start_idx(0, 0)
    hidx[1] = start_idx(1, 1)
    hw = start_w(0)
    hidx[0].wait()
    hg[0] = pltpu.async_copy(pt_hbm.at[idxb[0]], rb[0], sgb[0])
    for ch in range(NCH):
        sl = ch % 2
        if ch + 1 < NCH:
            hidx[1 - sl].wait()
            hg[1 - sl] = pltpu.async_copy(pt_hbm.at[idxb[1 - sl]],
                                          rb[1 - sl], sgb[1 - sl])
        hg[sl].wait()
        hw.wait()
        if hout is not None:
            hout.wait()
        compute(sl)
        hout = pltpu.async_copy(o0, z_hbm.at[pl.ds(n0 + ch * SUBQ, SUBQ)],
                                so0)
        if ch + 2 < NCH:
            hidx[sl] = start_idx(ch + 2, sl)
        if ch + 1 < NCH:
            hw = start_w(ch + 1)
    hout.wait()


def _interp_gather(pt, di_flat, dw_splat):
    mesh = plsc.VectorSubcoreMesh(core_axis_name="c", subcore_axis_name="s")
    return pl.kernel(
        _sc_body,
        mesh=mesh,
        out_type=jax.ShapeDtypeStruct((B * Q, D1), jnp.float32),
        scratch_types=(
            [pltpu.VMEM((3 * SUBQ,), jnp.int32) for _ in range(2)]
            + [pltpu.VMEM((SUBQ, 48), jnp.float32)]
            + [pltpu.VMEM((3 * SUBQ, D1), jnp.float32) for _ in range(2)]
            + [pltpu.VMEM((SUBQ, D1), jnp.float32)]
            + [pltpu.SemaphoreType.DMA for _ in range(6)]
        ),
    )(pt, di_flat, dw_splat)


def _kernel_c1(qf_ref, w1_ref, b1_ref, z_ref, y1_ref, s1_ref, ss1_ref):
    b = pl.program_id(0)
    qt = pl.program_id(1)
    yq = lax.dot_general(w1_ref[:, :Cq], qf_ref[0], (((1,), (0,)), ((), ())),
                         preferred_element_type=jnp.float32)
    y1 = yq + b1_ref[...] + jnp.transpose(z_ref[...])
    y1_ref[0] = y1

    @pl.when(jnp.logical_and(b == 0, qt == 0))
    def _():
        s1_ref[...] = jnp.zeros_like(s1_ref)
        ss1_ref[...] = jnp.zeros_like(ss1_ref)

    s1_ref[...] += jnp.sum(y1, axis=1, keepdims=True)
    ss1_ref[...] += jnp.sum(y1 * y1, axis=1, keepdims=True)


def _kernel_b(y1_ref, s1_ref, ss1_ref, g1_ref, be1_ref, w2_ref, b2_ref,
              y2_ref, s2_ref, ss2_ref):
    b = pl.program_id(0)
    qt = pl.program_id(1)

    mean = s1_ref[...] * (1.0 / N_BN)
    var = ss1_ref[...] * (1.0 / N_BN) - mean * mean
    scale = g1_ref[...] * lax.rsqrt(var + 1e-5)
    shift = be1_ref[...] - mean * scale
    x1 = jnp.maximum(y1_ref[0] * scale + shift, 0.0)
    y2 = lax.dot_general(w2_ref[...], x1, (((1,), (0,)), ((), ())),
                         preferred_element_type=jnp.float32) + b2_ref[...]
    y2_ref[0] = y2

    @pl.when(jnp.logical_and(b == 0, qt == 0))
    def _():
        s2_ref[...] = jnp.zeros_like(s2_ref)
        ss2_ref[...] = jnp.zeros_like(ss2_ref)

    s2_ref[...] += jnp.sum(y2, axis=1, keepdims=True)
    ss2_ref[...] += jnp.sum(y2 * y2, axis=1, keepdims=True)


def _kernel_c(y2_ref, s2_ref, ss2_ref, g2_ref, be2_ref, out_ref):
    mean = s2_ref[...] * (1.0 / N_BN)
    var = ss2_ref[...] * (1.0 / N_BN) - mean * mean
    scale = g2_ref[...] * lax.rsqrt(var + 1e-5)
    shift = be2_ref[...] - mean * scale
    out_ref[0] = jnp.maximum(y2_ref[0] * scale + shift, 0.0)


def kernel(q_points, s_points, q_feats, s_feats, W1, b1, g1, be1,
           W2, b2, g2, be2):
    qpt = q_points.transpose(0, 2, 1)
    b1c = b1.reshape(D1, 1)
    g1c = g1.reshape(D1, 1)
    be1c = be1.reshape(D1, 1)
    b2c = b2.reshape(D2, 1)
    g2c = g2.reshape(D2, 1)
    be2c = be2.reshape(D2, 1)

    col = lambda d: pl.BlockSpec((d, 1), lambda b, q: (0, 0))

    di, dw = pl.pallas_call(
        _kernel_a1,
        grid=(B, NQT),
        in_specs=[
            pl.BlockSpec((1, QT, 3), lambda b, q: (b, q, 0)),
            pl.BlockSpec((1, 3, S), lambda b, q: (b, 0, 0)),
        ],
        out_specs=[
            pl.BlockSpec((1, QT, 3), lambda b, q: (b, q, 0)),
            pl.BlockSpec((QT, 48), lambda b, q: (b * NQT + q, 0)),
        ],
        out_shape=[
            jax.ShapeDtypeStruct((B, Q, 3), jnp.int32),
            jax.ShapeDtypeStruct((B * Q, 48), jnp.float32),
        ],
    )(qpt, s_points)

    pt = pl.pallas_call(
        _kernel_a0,
        grid=(B,),
        in_specs=[
            pl.BlockSpec((1, Cs, S), lambda b: (b, 0, 0)),
            pl.BlockSpec((D1, D0), lambda b: (0, 0)),
        ],
        out_specs=pl.BlockSpec((S, D1), lambda b: (b, 0)),
        out_shape=jax.ShapeDtypeStruct((B * S, D1), jnp.float32),
    )(s_feats, W1)

    z = _interp_gather(pt, di.reshape(-1), dw)

    y1, s1, ss1 = pl.pallas_call(
        _kernel_c1,
        grid=(B, NQT2),
        in_specs=[
            pl.BlockSpec((1, Cq, QT2), lambda b, q: (b, 0, q)),
            pl.BlockSpec((D1, D0), lambda b, q: (0, 0)),
            col(D1),
            pl.BlockSpec((QT2, D1), lambda b, q: (b * NQT2 + q, 0)),
        ],
        out_specs=[
            pl.BlockSpec((1, D1, QT2), lambda b, q: (b, 0, q)),
            col(D1),
            col(D1),
        ],
        out_shape=[
            jax.ShapeDtypeStruct((B, D1, Q), jnp.float32),
            jax.ShapeDtypeStruct((D1, 1), jnp.float32),
            jax.ShapeDtypeStruct((D1, 1), jnp.float32),
        ],
    )(q_feats, W1, b1c, z)

    y2, s2, ss2 = pl.pallas_call(
        _kernel_b,
        grid=(B, NQT2),
        in_specs=[
            pl.BlockSpec((1, D1, QT2), lambda b, q: (b, 0, q)),
            col(D1), col(D1), col(D1), col(D1),
            pl.BlockSpec((D2, D1), lambda b, q: (0, 0)),
            col(D2),
        ],
        out_specs=[
            pl.BlockSpec((1, D2, QT2), lambda b, q: (b, 0, q)),
            col(D2),
            col(D2),
        ],
        out_shape=[
            jax.ShapeDtypeStruct((B, D2, Q), jnp.float32),
            jax.ShapeDtypeStruct((D2, 1), jnp.float32),
            jax.ShapeDtypeStruct((D2, 1), jnp.float32),
        ],
    )(y1, s1, ss1, g1c, be1c, W2, b2c)

    out = pl.pallas_call(
        _kernel_c,
        grid=(B, NQT2),
        in_specs=[
            pl.BlockSpec((1, D2, QT2), lambda b, q: (b, 0, q)),
            col(D2), col(D2), col(D2), col(D2),
        ],
        out_specs=pl.BlockSpec((1, D2, QT2), lambda b, q: (b, 0, q)),
        out_shape=jax.ShapeDtypeStruct((B, D2, Q), jnp.float32),
    )(y2, s2, ss2, g2c, be2c)
    return out
